# baseline (speedup 1.0000x reference)
import functools
import math

import jax
import jax.numpy as jnp
from jax import lax
from jax.experimental import pallas as pl
from jax.experimental.pallas import tpu as pltpu
from jax.experimental.pallas import tpu_sc as plsc

D_MODEL = 64
SCALE = math.sqrt(D_MODEL)
NC = 2
NS = 16
NW = NC * NS
CH = 128
LANES = 16
NBUF = 4


def _make_kernel(B):
    per_w = B // NW
    n_ch = per_w // CH
    n_grp = n_ch // NBUF
    mesh = plsc.VectorSubcoreMesh(core_axis_name="c", subcore_axis_name="s")

    scratch = [pltpu.VMEM((n_ch, CH), jnp.int32)]
    scratch += [pltpu.VMEM((CH, D_MODEL), jnp.float32) for _ in range(2 * NBUF)]
    scratch += [pltpu.SemaphoreType.DMA for _ in range(2 * NBUF)]

    @functools.partial(
        pl.kernel,
        mesh=mesh,
        compiler_params=pltpu.CompilerParams(use_tc_tiling_on_sc=False),
        out_type=jax.ShapeDtypeStruct((B, D_MODEL), jnp.float32),
        scratch_types=scratch,
    )
    def emb_kernel(x_hbm, tab_hbm, out_hbm, idx_v, *rest):
        g_bufs = rest[0:NBUF]
        s_bufs = rest[NBUF:2 * NBUF]
        g_sems = rest[2 * NBUF:3 * NBUF]
        o_sems = rest[3 * NBUF:4 * NBUF]
        wid = lax.axis_index("s") * NC + lax.axis_index("c")
        out_base = wid * per_w

        pltpu.sync_copy(x_hbm.at[pl.ds(wid * n_ch, n_ch)], idx_v)

        def start_gather(c, b):
            pltpu.async_copy(tab_hbm.at[idx_v.at[c]], g_bufs[b], g_sems[b])

        def wait_gather(b):
            pltpu.make_async_copy(
                tab_hbm.at[idx_v.at[0]], g_bufs[b], g_sems[b]).wait()

        def scale(b):
            def row(r, carry):
                for c in range(D_MODEL // LANES):
                    sl = pl.ds(c * LANES, LANES)
                    s_bufs[b][r, sl] = g_bufs[b][r, sl] * SCALE
                return carry
            lax.fori_loop(0, CH, row, 0)

        def start_out_g(c, b):
            pltpu.async_copy(
                g_bufs[b], out_hbm.at[pl.ds(out_base + c * CH, CH)], o_sems[b])

        def start_out(c, b):
            pltpu.async_copy(
                s_bufs[b], out_hbm.at[pl.ds(out_base + c * CH, CH)], o_sems[b])

        def wait_out(b):
            pltpu.make_async_copy(
                s_bufs[b], out_hbm.at[pl.ds(out_base, CH)], o_sems[b]).wait()

        for b in range(NBUF):
            start_gather(b, b)

        for b in range(NBUF):
            wait_gather(b)
            start_out_g(b, b)
            start_gather(NBUF + b, b)

        def group(j, carry):
            i = j * NBUF
            for b in range(NBUF):
                c = i + b
                wait_gather(b)
                wait_out(b)
                start_out_g(c, b)
                start_gather(c + NBUF, b)
            return carry

        lax.fori_loop(1, n_grp - 1, group, 0)

        i = (n_grp - 1) * NBUF
        for b in range(NBUF):
            c = i + b
            wait_gather(b)
            wait_out(b)
            start_out_g(c, b)

        for b in range(NBUF):
            wait_out(b)

    return emb_kernel


def kernel(x, table):
    S0, S1 = x.shape
    B = S0 * S1
    xf = x.reshape(B // CH, CH).astype(jnp.int32)
    out = _make_kernel(B)(xf, table)
    return out.reshape(S0, S1, D_MODEL)


# speedup vs baseline: 1.0030x; 1.0030x over previous
import functools
import math

import jax
import jax.numpy as jnp
from jax import lax
from jax.experimental import pallas as pl
from jax.experimental.pallas import tpu as pltpu
from jax.experimental.pallas import tpu_sc as plsc

D_MODEL = 64
SCALE = math.sqrt(D_MODEL)
NC = 2
NS = 16
NW = NC * NS
CH = 128
LANES = 16
NBUF = 4


def _make_kernel(B):
    per_w = B // NW
    n_ch = per_w // CH
    n_grp = n_ch // NBUF
    mesh = plsc.VectorSubcoreMesh(core_axis_name="c", subcore_axis_name="s")

    scratch = [pltpu.VMEM((n_ch, CH), jnp.int32)]
    scratch += [pltpu.VMEM((CH, D_MODEL), jnp.float32) for _ in range(2 * NBUF)]
    scratch += [pltpu.SemaphoreType.DMA for _ in range(2 * NBUF)]

    @functools.partial(
        pl.kernel,
        mesh=mesh,
        compiler_params=pltpu.CompilerParams(use_tc_tiling_on_sc=False),
        out_type=jax.ShapeDtypeStruct((B, D_MODEL), jnp.float32),
        scratch_types=scratch,
    )
    def emb_kernel(x_hbm, tab_hbm, out_hbm, idx_v, *rest):
        g_bufs = rest[0:NBUF]
        s_bufs = rest[NBUF:2 * NBUF]
        g_sems = rest[2 * NBUF:3 * NBUF]
        o_sems = rest[3 * NBUF:4 * NBUF]
        wid = lax.axis_index("s") * NC + lax.axis_index("c")
        out_base = wid * per_w

        pltpu.sync_copy(x_hbm.at[pl.ds(wid * n_ch, n_ch)], idx_v)

        def start_gather(c, b):
            pltpu.async_copy(tab_hbm.at[idx_v.at[c]], g_bufs[b], g_sems[b])

        def wait_gather(b):
            pltpu.make_async_copy(
                tab_hbm.at[idx_v.at[0]], g_bufs[b], g_sems[b]).wait()

        def scale(b):
            def row(r, carry):
                for c in range(D_MODEL // LANES):
                    sl = pl.ds(c * LANES, LANES)
                    s_bufs[b][r, sl] = g_bufs[b][r, sl] * SCALE
                return carry
            lax.fori_loop(0, CH, row, 0)

        def start_out(c, b):
            pltpu.async_copy(
                s_bufs[b], out_hbm.at[pl.ds(out_base + c * CH, CH)], o_sems[b])

        def wait_out(b):
            pltpu.make_async_copy(
                s_bufs[b], out_hbm.at[pl.ds(out_base, CH)], o_sems[b]).wait()

        for b in range(NBUF):
            start_gather(b, b)

        for b in range(NBUF):
            wait_gather(b)
            scale(b)
            start_out(b, b)
            start_gather(NBUF + b, b)

        def group(j, carry):
            i = j * NBUF
            for b in range(NBUF):
                c = i + b
                wait_gather(b)
                wait_out(b)
                scale(b)
                start_out(c, b)
                start_gather(c + NBUF, b)
            return carry

        lax.fori_loop(1, n_grp - 1, group, 0)

        i = (n_grp - 1) * NBUF
        for b in range(NBUF):
            c = i + b
            wait_gather(b)
            wait_out(b)
            scale(b)
            start_out(c, b)

        for b in range(NBUF):
            wait_out(b)

    return emb_kernel


def kernel(x, table):
    S0, S1 = x.shape
    B = S0 * S1
    xf = x.reshape(B // CH, CH).astype(jnp.int32)
    out = _make_kernel(B)(xf, table)
    return out.reshape(S0, S1, D_MODEL)
